# Initial kernel scaffold; baseline (speedup 1.0000x reference)
import jax
import jax.numpy as jnp
from jax.experimental import pallas as pl


def kernel(logits):
    raise NotImplementedError("write your pallas kernel here")


# speedup vs baseline: 1.3637x; 1.3637x over previous
import jax
import jax.numpy as jnp
from jax.experimental import pallas as pl

_THRESH = 0.02


def _body(x_ref, out_ref, eh_ref):
    x = x_ref[...]
    Hn = x.shape[0]
    m = jnp.max(x, axis=-1, keepdims=True)
    s = jnp.sum(jnp.exp(x - m), axis=-1)
    conf = 1.0 / s
    ex = conf >= jnp.float32(_THRESH)
    eh = jnp.full(x.shape[1:2], Hn - 1, jnp.int32)
    out = x[Hn - 1]
    for h in range(Hn - 2, -1, -1):
        eh = jnp.where(ex[h], jnp.int32(h), eh)
        out = jnp.where((eh == h)[:, None], x[h], out)
    out_ref[...] = out
    eh_ref[...] = eh


def kernel(logits):
    Hn, Bn, Cn = logits.shape
    BB = 512
    grid = (Bn // BB,)
    out, eh = pl.pallas_call(
        _body,
        grid=grid,
        in_specs=[pl.BlockSpec((Hn, BB, Cn), lambda i: (0, i, 0))],
        out_specs=[
            pl.BlockSpec((BB, Cn), lambda i: (i, 0)),
            pl.BlockSpec((BB,), lambda i: (i,)),
        ],
        out_shape=[
            jax.ShapeDtypeStruct((Bn, Cn), logits.dtype),
            jax.ShapeDtypeStruct((Bn,), jnp.int32),
        ],
    )(logits)
    return out, eh
